# baseline (speedup 1.0000x reference)
import jax
import jax.numpy as jnp
from jax import lax
from jax.experimental import pallas as pl
from jax.experimental.pallas import tpu as pltpu
from jax.experimental.pallas import tpu_sc as plsc

N_NODES = 10000
N_EDGES = 320000
D = 128
BN_EPS = 1e-5
LEAKY_SLOPE = 0.01

NC = 2
NS = 16
NW = NC * NS
CHUNK = 88
CPB = 8
NBLK = 15
CPW = NBLK * CPB
NBUF = 4
E_PAD = NW * CPW * CHUNK
N_PAD = 10112
ROWS_PER_TILE = N_PAD // NS


def _sc_scatter_sum(h, src, dst, zinit):
    def body(h_hbm, src_hbm, dst_hbm, z_hbm, out_hbm,
             idx_s, idx_d, rows, sem_g, sem_s, aggr):
        c = lax.axis_index("c")
        s = lax.axis_index("s")
        r0 = s * ROWS_PER_TILE
        pltpu.sync_copy(z_hbm.at[pl.ds(r0, ROWS_PER_TILE)],
                        aggr.at[pl.ds(r0, ROWS_PER_TILE)])
        plsc.subcore_barrier()

        def fire_g(t, buf):
            pltpu.async_copy(h_hbm.at[idx_s.at[t]], rows[buf], sem_g[buf])

        def wait_g(buf):
            pltpu.make_async_copy(
                h_hbm.at[idx_s.at[0]], rows[buf], sem_g[buf]).wait()

        def fire_s(t, buf):
            pltpu.async_copy(rows[buf], aggr.at[idx_d.at[t]], sem_s[buf],
                             add=True)

        def wait_s(buf):
            pltpu.make_async_copy(
                rows[buf], aggr.at[idx_d.at[0]], sem_s[buf]).wait()

        def blk_body(b, carry):
            b0 = pl.multiple_of(b * CPB, CPB)
            pltpu.sync_copy(src_hbm.at[c, s, pl.ds(b0, CPB)], idx_s)
            pltpu.sync_copy(dst_hbm.at[c, s, pl.ds(b0, CPB)], idx_d)
            for t in (0, 1):
                @pl.when(b > 0)
                def _(t=t):
                    wait_s(t)
                fire_g(t, t)
            for t in range(CPB):
                buf = t % NBUF
                wait_g(buf)
                fire_s(t, buf)
                if t < CPB - 2:
                    nbuf = (t + 2) % NBUF
                    if t < 2:
                        @pl.when(b > 0)
                        def _(nbuf=nbuf):
                            wait_s(nbuf)
                    else:
                        wait_s(nbuf)
                    fire_g(t + 2, nbuf)
            return carry

        lax.fori_loop(0, NBLK, blk_body, 0)
        for buf in range(NBUF):
            wait_s(buf)
        plsc.subcore_barrier()
        pltpu.sync_copy(aggr.at[pl.ds(r0, ROWS_PER_TILE)],
                        out_hbm.at[c, pl.ds(r0, ROWS_PER_TILE)])

    mesh = plsc.VectorSubcoreMesh(core_axis_name="c", subcore_axis_name="s")
    run = pl.kernel(
        body,
        out_type=jax.ShapeDtypeStruct((NC, N_PAD, D), jnp.float32),
        mesh=mesh,
        scratch_types=[
            pltpu.VMEM((CPB, CHUNK), jnp.int32),
            pltpu.VMEM((CPB, CHUNK), jnp.int32),
            [pltpu.VMEM((CHUNK, D), jnp.float32) for _ in range(NBUF)],
            [pltpu.SemaphoreType.DMA for _ in range(NBUF)],
            [pltpu.SemaphoreType.DMA for _ in range(NBUF)],
            pltpu.VMEM_SHARED((N_PAD, D), jnp.float32),
        ],
    )
    return run(h, src, dst, zinit)


def _tc_body(h_ref, p0_ref, p1_ref, w1_ref, b1_ref, w2_ref, b2_ref,
             g_ref, bt_ref, out_ref):
    h = h_ref[...]
    rst = h + p0_ref[...] + p1_ref[...]
    z = jnp.maximum(
        jnp.dot(rst, w1_ref[...], preferred_element_type=jnp.float32)
        + b1_ref[...], 0.0)
    z = jnp.dot(z, w2_ref[...], preferred_element_type=jnp.float32) + b2_ref[...]
    mean = jnp.mean(z, axis=0, keepdims=True)
    d = z - mean
    var = jnp.mean(d * d, axis=0, keepdims=True)
    zn = d * lax.rsqrt(var + BN_EPS) * g_ref[...] + bt_ref[...]
    zn = jnp.where(zn >= 0, zn, LEAKY_SLOPE * zn)
    out_ref[...] = h + zn


def kernel(h, edge_index, W1, b1, W2, b2, gamma, beta):
    src = edge_index[0].astype(jnp.int32)
    dst = edge_index[1].astype(jnp.int32)
    pad = E_PAD - N_EDGES
    src = jnp.concatenate([src, jnp.zeros((pad,), jnp.int32)])
    dst = jnp.concatenate([dst, jnp.full((pad,), N_NODES, jnp.int32)])
    src = src.reshape(NC, NS, CPW, CHUNK)
    dst = dst.reshape(NC, NS, CPW, CHUNK)
    zinit = jnp.zeros((N_PAD, D), jnp.float32)

    partials = _sc_scatter_sum(h, src, dst, zinit)
    p0 = partials[0, :N_NODES]
    p1 = partials[1, :N_NODES]

    out = pl.pallas_call(
        _tc_body,
        out_shape=jax.ShapeDtypeStruct((N_NODES, D), jnp.float32),
    )(h, p0, p1, W1, b1.reshape(1, D), W2, b2.reshape(1, D),
      gamma.reshape(1, D), beta.reshape(1, D))
    return out
